# baseline (speedup 1.0000x reference)
import functools

import jax
import jax.numpy as jnp
from jax import lax
from jax.experimental import pallas as pl
from jax.experimental.pallas import tpu as pltpu
from jax.experimental.pallas import tpu_sc as plsc

D = 128
H = 64
NC = 2
NS = 16
G = 32


def _seg_mesh():
    return plsc.VectorSubcoreMesh(core_axis_name="c", subcore_axis_name="s",
                                  num_cores=NC, num_subcores=NS)


R = 2000


def _seg_sum(tables, srcs, dst, num_out, block, r=2000, g=32, nbuf=2, la=1):
    E = dst.shape[0]
    T = len(tables)
    ept = E // NS
    assert E % (NS * 16) == 0 and ept % r == 0
    assert block % (NS * 8) == 0
    assert 1 <= la <= nbuf - 1
    nrounds = ept // r
    nb = -(-num_out // block)
    rpt = block // NS
    out_rows = nb * block
    src_arrs = [s for s in srcs if not isinstance(s, str)]
    NA = len(src_arrs)
    src_map = []
    _na = 0
    for s_ in srcs:
        if isinstance(s_, str):
            src_map.append(None)
        else:
            src_map.append(_na)
            _na += 1
    cap = r + g

    def body(*refs):
        it = iter(refs)
        tabs = [next(it) for _ in range(T)]
        sarr = [next(it) for _ in range(NA)]
        dst_h = next(it)
        zeros_h = next(it)
        outs = [next(it) for _ in range(T)]
        dst_e = next(it)
        src_e = [next(it) for _ in range(NA)]
        stage_rel = next(it)
        stages = [next(it) for _ in range(T)]
        rows_v = [next(it) for _ in range(T)]
        accs = [next(it) for _ in range(T)]
        sem_e = next(it)
        sem_g = [next(it) for _ in range(T)]
        sem_s = [next(it) for _ in range(T)]

        c = lax.axis_index("c")
        s = lax.axis_index("s")
        base_e = s * ept

        def zero_accs():
            for acc in accs:
                pltpu.sync_copy(zeros_h.at[pl.ds(0, rpt)],
                                acc.at[pl.ds(s * rpt, rpt)])

                @pl.when(s == NS - 1)
                def _():
                    pltpu.sync_copy(zeros_h.at[pl.ds(0, 8)],
                                    acc.at[pl.ds(block, 8)])

        zero_accs()
        plsc.subcore_barrier()

        iota16 = lax.iota(jnp.int32, 16)

        def issue_edges(rr):
            slot = lax.rem(rr, 2)
            pltpu.async_copy(dst_h.at[pl.ds(base_e + rr * r, r)],
                             dst_e.at[pl.ds(slot * r, r)], sem_e)
            for a in range(NA):
                pltpu.async_copy(sarr[a].at[pl.ds(base_e + rr * r, r)],
                                 src_e[a].at[pl.ds(slot * r, r)], sem_e)

        def wait_edges():
            for _ in range(1 + NA):
                pltpu.make_async_copy(dst_h.at[pl.ds(0, r)],
                                      dst_e.at[pl.ds(0, r)], sem_e).wait()

        def issue_gather(t, j, slot):
            idx = stages[t].at[pl.ds(j * g, g)]
            pltpu.async_copy(tabs[t].at[idx], rows_v[t].at[slot], sem_g[t])

        def wait_gather(t):
            pltpu.make_async_copy(tabs[t].at[pl.ds(0, g)],
                                  rows_v[t].at[0], sem_g[t]).wait()

        def issue_scatter(t, j, slot):
            idx = stage_rel.at[pl.ds(j * g, g)]
            pltpu.async_copy(rows_v[t].at[slot], accs[t].at[idx], sem_s[t],
                             add=True)

        def wait_scatter(t):
            pltpu.make_async_copy(rows_v[t].at[0],
                                  accs[t].at[pl.ds(0, g)], sem_s[t]).wait()

        def per_block(b, carry):
            @pl.when(lax.rem(b, NC) == c)
            def _():
                lo = b * block
                issue_edges(0)

                def per_round(rr, carry1):
                    wait_edges()

                    @pl.when(rr + 1 < nrounds)
                    def _():
                        issue_edges(rr + 1)

                    eslot = lax.rem(rr, 2) * r
                    ebase = base_e + rr * r

                    def scan_body(i, wp):
                        d = dst_e[pl.ds(eslot + i * 16, 16)]
                        rel = d - lo
                        m = plsc.bitcast(rel, jnp.uint32) < jnp.uint32(block)
                        plsc.store_compressed(stage_rel.at[pl.ds(wp, 16)],
                                              rel, mask=m)
                        for t in range(T):
                            if src_map[t] is None:
                                sv = (ebase + i * 16) + iota16
                            else:
                                sv = src_e[src_map[t]][
                                    pl.ds(eslot + i * 16, 16)]
                            plsc.store_compressed(
                                stages[t].at[pl.ds(wp, 16)], sv, mask=m)
                        cnt = plsc.all_reduce_population_count(m)
                        return wp + cnt[0]

                    n = lax.fori_loop(0, r // 16, scan_body,
                                      jnp.int32(0))
                    for k in range(g // 16):
                        stage_rel[pl.ds(n + k * 16, 16)] = (
                            jnp.full((16,), block, jnp.int32))
                        for t in range(T):
                            stages[t][pl.ds(n + k * 16, 16)] = iota16

                    nf = (n + g) // g

                    for k in range(la):
                        @pl.when(nf > k)
                        def _(k=k):
                            for t in range(T):
                                issue_gather(t, k, k % nbuf)

                    def flush_body(j, carry2):
                        slot = lax.rem(j, nbuf)

                        @pl.when(j >= nbuf - la)
                        def _():
                            for t in range(T):
                                wait_scatter(t)

                        @pl.when(j + la < nf)
                        def _():
                            for t in range(T):
                                issue_gather(t, j + la,
                                             lax.rem(j + la, nbuf))
                        for t in range(T):
                            wait_gather(t)
                            issue_scatter(t, j, slot)
                        return carry2

                    lax.fori_loop(0, nf, flush_body, 0)
                    for t in range(T):
                        wait_scatter(t)
                    for k in range(2, nbuf - la + 1):
                        @pl.when(nf >= k)
                        def _(k=k):
                            for t in range(T):
                                wait_scatter(t)
                    return carry1

                lax.fori_loop(0, nrounds, per_round, 0)

                plsc.subcore_barrier()
                for t in range(T):
                    pltpu.sync_copy(accs[t].at[pl.ds(s * rpt, rpt)],
                                    outs[t].at[pl.ds(lo + s * rpt, rpt)])
                plsc.subcore_barrier()
                zero_accs()
                plsc.subcore_barrier()
            return carry

        lax.fori_loop(0, nb, per_block, 0)

    zeros_h = jnp.zeros((rpt, D), jnp.float32)
    fn = pl.kernel(
        body,
        out_type=[jax.ShapeDtypeStruct((out_rows, D), jnp.float32)
                  for _ in range(T)],
        mesh=_seg_mesh(),
        scratch_types=(
            [pltpu.VMEM((2 * r,), jnp.int32)]
            + [pltpu.VMEM((2 * r,), jnp.int32) for _ in range(NA)]
            + [pltpu.VMEM((cap,), jnp.int32)]
            + [pltpu.VMEM((cap,), jnp.int32) for _ in range(T)]
            + [pltpu.VMEM((nbuf, g, D), jnp.float32) for _ in range(T)]
            + [pltpu.VMEM_SHARED((block + 8, D), jnp.float32)
               for _ in range(T)]
            + [pltpu.SemaphoreType.DMA]
            + [pltpu.SemaphoreType.DMA for _ in range(2 * T)]
        ),
        compiler_params=pltpu.CompilerParams(needs_layout_passes=False),
    )
    out = fn(*tables, *src_arrs, dst, zeros_h)
    return out if isinstance(out, (list, tuple)) else [out]


def _gather_rows(table, idx):
    B = idx.shape[0]
    GG = 200
    per_w = B // (NC * NS)
    assert per_w % GG == 0
    nf = per_w // GG

    def body(table_h, idx_h, out_h, idx_v, rows_v, sem_g, sem_o):
        c = lax.axis_index("c")
        s = lax.axis_index("s")
        wid = s * NC + c
        base = wid * per_w
        pltpu.sync_copy(idx_h.at[pl.ds(base, per_w)], idx_v)

        def issue_gather(j, slot):
            ii = idx_v.at[pl.ds(j * GG, GG)]
            pltpu.async_copy(table_h.at[ii], rows_v.at[slot], sem_g)

        issue_gather(0, 0)

        def step(j, carry):
            slot = lax.rem(j, 2)

            @pl.when(j >= 1)
            def _():
                pltpu.make_async_copy(rows_v.at[0],
                                      out_h.at[pl.ds(0, GG)], sem_o).wait()

            @pl.when(j + 1 < nf)
            def _():
                issue_gather(j + 1, 1 - slot)
            pltpu.make_async_copy(table_h.at[pl.ds(0, GG)],
                                  rows_v.at[0], sem_g).wait()
            pltpu.async_copy(rows_v.at[slot],
                             out_h.at[pl.ds(base + j * GG, GG)], sem_o)
            return carry

        lax.fori_loop(0, nf, step, 0)
        pltpu.make_async_copy(rows_v.at[0],
                              out_h.at[pl.ds(0, GG)], sem_o).wait()

    fn = pl.kernel(
        body,
        out_type=jax.ShapeDtypeStruct((B, D), jnp.float32),
        mesh=_seg_mesh(),
        scratch_types=[
            pltpu.VMEM((per_w,), jnp.int32),
            pltpu.VMEM((2, GG, D), jnp.float32),
            pltpu.SemaphoreType.DMA,
            pltpu.SemaphoreType.DMA,
        ],
        compiler_params=pltpu.CompilerParams(needs_layout_passes=False),
    )
    return fn(table, idx)


def _fused_body(x_r, dg_r, a2_r, a3_r, a4_r, w0, w1, w2, w3, w4, bs,
                out_r, st_r):
    i = pl.program_id(0)
    bf = jnp.bfloat16
    xb = x_r[...]
    acc = jnp.dot(xb.astype(bf), w0[...], preferred_element_type=jnp.float32)
    acc += jnp.dot((dg_r[...] * xb).astype(bf), w1[...],
                   preferred_element_type=jnp.float32)
    acc += jnp.dot(a2_r[...].astype(bf), w2[...],
                   preferred_element_type=jnp.float32)
    acc += jnp.dot(a3_r[...].astype(bf), w3[...],
                   preferred_element_type=jnp.float32)
    acc += jnp.dot(a4_r[...].astype(bf), w4[...],
                   preferred_element_type=jnp.float32)
    acc += bs[...]
    col = lax.broadcasted_iota(jnp.int32, (1, D), 1)
    cat = jnp.where(col < H, acc, jnp.maximum(acc, 0.0))
    out_r[...] = cat.astype(out_r.dtype)

    @pl.when(i == 0)
    def _():
        st_r[...] = jnp.zeros_like(st_r)

    su = jnp.sum(cat, axis=0, keepdims=True)
    sq = jnp.sum(cat * cat, axis=0, keepdims=True)
    st_r[...] += jnp.concatenate([su, sq, jnp.zeros((6, D), jnp.float32)], 0)


def _fused_pass(x, deg, a2, a3, a4, w0, w1, w2, w3, w4, bsum, rt,
                out_dtype=jnp.float32):
    n = x.shape[0]
    assert n % rt == 0
    grid = n // rt
    row = lambda i: (i, 0)
    fix = lambda i: (0, 0)
    return pl.pallas_call(
        _fused_body,
        grid=(grid,),
        in_specs=[
            pl.BlockSpec((rt, D), row),
            pl.BlockSpec((rt, 1), row),
            pl.BlockSpec((rt, D), row),
            pl.BlockSpec((rt, D), row),
            pl.BlockSpec((rt, D), row),
            pl.BlockSpec((D, D), fix),
            pl.BlockSpec((D, D), fix),
            pl.BlockSpec((D, D), fix),
            pl.BlockSpec((D, D), fix),
            pl.BlockSpec((D, D), fix),
            pl.BlockSpec((1, D), fix),
        ],
        out_specs=[
            pl.BlockSpec((rt, D), row),
            pl.BlockSpec((8, D), fix),
        ],
        out_shape=[
            jax.ShapeDtypeStruct((n, D), out_dtype),
            jax.ShapeDtypeStruct((8, D), jnp.float32),
        ],
    )(x, deg, a2, a3, a4,
      w0.astype(jnp.bfloat16), w1.astype(jnp.bfloat16),
      w2.astype(jnp.bfloat16), w3.astype(jnp.bfloat16),
      w4.astype(jnp.bfloat16), bsum)


def _bn_body(cnt, cat_r, st_r, w_r, b_r, out_r):
    mu = st_r[0:1, :] / cnt
    var = st_r[1:2, :] / cnt - mu * mu
    inv = lax.rsqrt(var + 1e-5)
    cat = cat_r[...].astype(jnp.float32)
    out_r[...] = (cat - mu) * inv * w_r[...] + b_r[...]


def _bn_pass(cat, stats, w, b, rt):
    n = cat.shape[0]
    grid = n // rt
    row = lambda i: (i, 0)
    fix = lambda i: (0, 0)
    return pl.pallas_call(
        functools.partial(_bn_body, float(n)),
        grid=(grid,),
        in_specs=[
            pl.BlockSpec((rt, D), row),
            pl.BlockSpec((8, D), fix),
            pl.BlockSpec((1, D), fix),
            pl.BlockSpec((1, D), fix),
        ],
        out_specs=pl.BlockSpec((rt, D), row),
        out_shape=jax.ShapeDtypeStruct((n, D), jnp.float32),
    )(cat, stats, w.reshape(1, D), b.reshape(1, D))


def kernel(x, y, deg_g, deg_lg, pm_pd, edge_index_g, edge_index_lg,
           Wtx, btx, Wtd, btd, Wty, bty, Wt0, bt0, Wt1, bt1,
           Wgy, bgy, Wgd, bgd, Wgx, bgx, Wg0, bg0, Wg1, bg1,
           bnx_w, bnx_b, bny_w, bny_b):
    n = x.shape[0]
    m = y.shape[0]

    src_g, dst_g = edge_index_g[0], edge_index_g[1]
    src_l, dst_l = edge_index_lg[0], edge_index_lg[1]

    zb = 5120
    z1, pmpd_y = _seg_sum([x, y], [src_g, "edge"], dst_g, n, zb,
                          r=2000, g=32, nbuf=3, la=2)
    (z2p,) = _seg_sum([z1], [src_g], dst_g, n, zb,
                      r=4000, g=64, nbuf=4, la=2)
    wb = 12800
    (w1,) = _seg_sum([y], [src_l], dst_l, m, wb,
                     r=2000, g=32, nbuf=3, la=2)
    (w2,) = _seg_sum([w1], [src_l], dst_l, m, wb,
                     r=2000, g=32, nbuf=3, la=2)
    pmpd_x = _gather_rows(x, pm_pd)

    bsx = (btx + btd + bt0 + bt1 + bty).reshape(1, D)
    xcat, xst = _fused_pass(x, deg_g, z1[:n], z2p[:n], pmpd_y[:n],
                            Wtx.T, Wtd.T, Wt0.T, Wt1.T, Wty.T, bsx, 1000)
    xn = _bn_pass(xcat, xst, bnx_w, bnx_b, 1000)

    bsy = (bgy + bgd + bg0 + bg1 + bgx).reshape(1, D)
    ycat, yst = _fused_pass(y, deg_lg, w1, w2, pmpd_x,
                            Wgy.T, Wgd.T, Wg0.T, Wg1.T, Wgx.T, bsy, 2000,
                            out_dtype=jnp.bfloat16)
    yn = _bn_pass(ycat, yst, bny_w, bny_b, 2000)
    return (xn, yn)


# speedup vs baseline: 1.0185x; 1.0185x over previous
import functools

import jax
import jax.numpy as jnp
from jax import lax
from jax.experimental import pallas as pl
from jax.experimental.pallas import tpu as pltpu
from jax.experimental.pallas import tpu_sc as plsc

D = 128
H = 64
NC = 2
NS = 16
G = 32


def _seg_mesh():
    return plsc.VectorSubcoreMesh(core_axis_name="c", subcore_axis_name="s",
                                  num_cores=NC, num_subcores=NS)


R = 2000


def _seg_sum(tables, srcs, dst, num_out, block, r=2000, g=32, nbuf=2, la=1):
    E = dst.shape[0]
    T = len(tables)
    ept = E // NS
    assert E % (NS * 16) == 0 and ept % r == 0
    assert block % (NS * 8) == 0
    assert 1 <= la <= nbuf - 1
    nrounds = ept // r
    nb = -(-num_out // block)
    rpt = block // NS
    out_rows = nb * block
    src_arrs = [s for s in srcs if not isinstance(s, str)]
    NA = len(src_arrs)
    src_map = []
    _na = 0
    for s_ in srcs:
        if isinstance(s_, str):
            src_map.append(None)
        else:
            src_map.append(_na)
            _na += 1
    cap = r + g

    def body(*refs):
        it = iter(refs)
        tabs = [next(it) for _ in range(T)]
        sarr = [next(it) for _ in range(NA)]
        dst_h = next(it)
        zeros_h = next(it)
        outs = [next(it) for _ in range(T)]
        dst_e = next(it)
        src_e = [next(it) for _ in range(NA)]
        stage_rel = next(it)
        stages = [next(it) for _ in range(T)]
        rows_v = [next(it) for _ in range(T)]
        accs = [next(it) for _ in range(T)]
        sem_e = next(it)
        sem_g = [next(it) for _ in range(T)]
        sem_s = [next(it) for _ in range(T)]

        c = lax.axis_index("c")
        s = lax.axis_index("s")
        base_e = s * ept

        def zero_accs():
            for acc in accs:
                pltpu.sync_copy(zeros_h.at[pl.ds(0, rpt)],
                                acc.at[pl.ds(s * rpt, rpt)])

                @pl.when(s == NS - 1)
                def _():
                    pltpu.sync_copy(zeros_h.at[pl.ds(0, 8)],
                                    acc.at[pl.ds(block, 8)])

        zero_accs()
        plsc.subcore_barrier()

        iota16 = lax.iota(jnp.int32, 16)

        def issue_edges(rr):
            slot = lax.rem(rr, 2)
            pltpu.async_copy(dst_h.at[pl.ds(base_e + rr * r, r)],
                             dst_e.at[pl.ds(slot * r, r)], sem_e)
            for a in range(NA):
                pltpu.async_copy(sarr[a].at[pl.ds(base_e + rr * r, r)],
                                 src_e[a].at[pl.ds(slot * r, r)], sem_e)

        def wait_edges():
            for _ in range(1 + NA):
                pltpu.make_async_copy(dst_h.at[pl.ds(0, r)],
                                      dst_e.at[pl.ds(0, r)], sem_e).wait()

        def issue_gather(t, j, slot):
            idx = stages[t].at[pl.ds(j * g, g)]
            pltpu.async_copy(tabs[t].at[idx], rows_v[t].at[slot], sem_g[t])

        def wait_gather(t):
            pltpu.make_async_copy(tabs[t].at[pl.ds(0, g)],
                                  rows_v[t].at[0], sem_g[t]).wait()

        def issue_scatter(t, j, slot):
            idx = stage_rel.at[pl.ds(j * g, g)]
            pltpu.async_copy(rows_v[t].at[slot], accs[t].at[idx], sem_s[t],
                             add=True)

        def wait_scatter(t):
            pltpu.make_async_copy(rows_v[t].at[0],
                                  accs[t].at[pl.ds(0, g)], sem_s[t]).wait()

        def per_block(b, carry):
            @pl.when(lax.rem(b, NC) == c)
            def _():
                lo = b * block
                issue_edges(0)

                def per_round(rr, carry1):
                    wait_edges()

                    @pl.when(rr + 1 < nrounds)
                    def _():
                        issue_edges(rr + 1)

                    eslot = lax.rem(rr, 2) * r
                    ebase = base_e + rr * r

                    def scan_body(i, wp):
                        d = dst_e[pl.ds(eslot + i * 16, 16)]
                        rel = d - lo
                        m = plsc.bitcast(rel, jnp.uint32) < jnp.uint32(block)
                        plsc.store_compressed(stage_rel.at[pl.ds(wp, 16)],
                                              rel, mask=m)
                        for t in range(T):
                            if src_map[t] is None:
                                sv = (ebase + i * 16) + iota16
                            else:
                                sv = src_e[src_map[t]][
                                    pl.ds(eslot + i * 16, 16)]
                            plsc.store_compressed(
                                stages[t].at[pl.ds(wp, 16)], sv, mask=m)
                        cnt = plsc.all_reduce_population_count(m)
                        return wp + cnt[0]

                    n = lax.fori_loop(0, r // 16, scan_body,
                                      jnp.int32(0))
                    for k in range(g // 16):
                        stage_rel[pl.ds(n + k * 16, 16)] = (
                            jnp.full((16,), block, jnp.int32))
                        for t in range(T):
                            stages[t][pl.ds(n + k * 16, 16)] = iota16

                    nf = (n + g) // g

                    for k in range(la):
                        @pl.when(nf > k)
                        def _(k=k):
                            for t in range(T):
                                issue_gather(t, k, k % nbuf)

                    def flush_body(j, carry2):
                        slot = lax.rem(j, nbuf)

                        @pl.when(j >= nbuf - la)
                        def _():
                            for t in range(T):
                                wait_scatter(t)

                        @pl.when(j + la < nf)
                        def _():
                            for t in range(T):
                                issue_gather(t, j + la,
                                             lax.rem(j + la, nbuf))
                        for t in range(T):
                            wait_gather(t)
                            issue_scatter(t, j, slot)
                        return carry2

                    lax.fori_loop(0, nf, flush_body, 0)
                    for t in range(T):
                        wait_scatter(t)
                    for k in range(2, nbuf - la + 1):
                        @pl.when(nf >= k)
                        def _(k=k):
                            for t in range(T):
                                wait_scatter(t)
                    return carry1

                lax.fori_loop(0, nrounds, per_round, 0)

                plsc.subcore_barrier()
                for t in range(T):
                    pltpu.sync_copy(accs[t].at[pl.ds(s * rpt, rpt)],
                                    outs[t].at[pl.ds(lo + s * rpt, rpt)])
                plsc.subcore_barrier()
                zero_accs()
                plsc.subcore_barrier()
            return carry

        lax.fori_loop(0, nb, per_block, 0)

    zeros_h = jnp.zeros((rpt, D), jnp.float32)
    fn = pl.kernel(
        body,
        out_type=[jax.ShapeDtypeStruct((out_rows, D), jnp.float32)
                  for _ in range(T)],
        mesh=_seg_mesh(),
        scratch_types=(
            [pltpu.VMEM((2 * r,), jnp.int32)]
            + [pltpu.VMEM((2 * r,), jnp.int32) for _ in range(NA)]
            + [pltpu.VMEM((cap,), jnp.int32)]
            + [pltpu.VMEM((cap,), jnp.int32) for _ in range(T)]
            + [pltpu.VMEM((nbuf, g, D), jnp.float32) for _ in range(T)]
            + [pltpu.VMEM_SHARED((block + 8, D), jnp.float32)
               for _ in range(T)]
            + [pltpu.SemaphoreType.DMA]
            + [pltpu.SemaphoreType.DMA for _ in range(2 * T)]
        ),
        compiler_params=pltpu.CompilerParams(needs_layout_passes=False),
    )
    out = fn(*tables, *src_arrs, dst, zeros_h)
    return out if isinstance(out, (list, tuple)) else [out]


def _gather_rows(table, idx):
    B = idx.shape[0]
    GG = 200
    per_w = B // (NC * NS)
    assert per_w % GG == 0
    nf = per_w // GG

    def body(table_h, idx_h, out_h, idx_v, rows_v, sem_g, sem_o):
        c = lax.axis_index("c")
        s = lax.axis_index("s")
        wid = s * NC + c
        base = wid * per_w
        pltpu.sync_copy(idx_h.at[pl.ds(base, per_w)], idx_v)

        def issue_gather(j, slot):
            ii = idx_v.at[pl.ds(j * GG, GG)]
            pltpu.async_copy(table_h.at[ii], rows_v.at[slot], sem_g)

        issue_gather(0, 0)

        def step(j, carry):
            slot = lax.rem(j, 2)

            @pl.when(j >= 1)
            def _():
                pltpu.make_async_copy(rows_v.at[0],
                                      out_h.at[pl.ds(0, GG)], sem_o).wait()

            @pl.when(j + 1 < nf)
            def _():
                issue_gather(j + 1, 1 - slot)
            pltpu.make_async_copy(table_h.at[pl.ds(0, GG)],
                                  rows_v.at[0], sem_g).wait()
            pltpu.async_copy(rows_v.at[slot],
                             out_h.at[pl.ds(base + j * GG, GG)], sem_o)
            return carry

        lax.fori_loop(0, nf, step, 0)
        pltpu.make_async_copy(rows_v.at[0],
                              out_h.at[pl.ds(0, GG)], sem_o).wait()

    fn = pl.kernel(
        body,
        out_type=jax.ShapeDtypeStruct((B, D), jnp.float32),
        mesh=_seg_mesh(),
        scratch_types=[
            pltpu.VMEM((per_w,), jnp.int32),
            pltpu.VMEM((2, GG, D), jnp.float32),
            pltpu.SemaphoreType.DMA,
            pltpu.SemaphoreType.DMA,
        ],
        compiler_params=pltpu.CompilerParams(needs_layout_passes=False),
    )
    return fn(table, idx)


def _fused_body(x_r, dg_r, a2_r, a3_r, a4_r, w0, w1, w2, w3, w4, bs,
                out_r, st_r):
    i = pl.program_id(0)
    xb = x_r[...]
    acc = jnp.dot(xb, w0[...], preferred_element_type=jnp.float32)
    acc += jnp.dot(dg_r[...] * xb, w1[...], preferred_element_type=jnp.float32)
    acc += jnp.dot(a2_r[...], w2[...], preferred_element_type=jnp.float32)
    acc += jnp.dot(a3_r[...], w3[...], preferred_element_type=jnp.float32)
    acc += jnp.dot(a4_r[...], w4[...], preferred_element_type=jnp.float32)
    acc += bs[...]
    col = lax.broadcasted_iota(jnp.int32, (1, D), 1)
    cat = jnp.where(col < H, acc, jnp.maximum(acc, 0.0))
    out_r[...] = cat.astype(out_r.dtype)

    @pl.when(i == 0)
    def _():
        st_r[...] = jnp.zeros_like(st_r)

    su = jnp.sum(cat, axis=0, keepdims=True)
    sq = jnp.sum(cat * cat, axis=0, keepdims=True)
    st_r[...] += jnp.concatenate([su, sq, jnp.zeros((6, D), jnp.float32)], 0)


def _fused_pass(x, deg, a2, a3, a4, w0, w1, w2, w3, w4, bsum, rt,
                out_dtype=jnp.float32):
    n = x.shape[0]
    assert n % rt == 0
    grid = n // rt
    row = lambda i: (i, 0)
    fix = lambda i: (0, 0)
    return pl.pallas_call(
        _fused_body,
        grid=(grid,),
        in_specs=[
            pl.BlockSpec((rt, D), row),
            pl.BlockSpec((rt, 1), row),
            pl.BlockSpec((rt, D), row),
            pl.BlockSpec((rt, D), row),
            pl.BlockSpec((rt, D), row),
            pl.BlockSpec((D, D), fix),
            pl.BlockSpec((D, D), fix),
            pl.BlockSpec((D, D), fix),
            pl.BlockSpec((D, D), fix),
            pl.BlockSpec((D, D), fix),
            pl.BlockSpec((1, D), fix),
        ],
        out_specs=[
            pl.BlockSpec((rt, D), row),
            pl.BlockSpec((8, D), fix),
        ],
        out_shape=[
            jax.ShapeDtypeStruct((n, D), out_dtype),
            jax.ShapeDtypeStruct((8, D), jnp.float32),
        ],
    )(x, deg, a2, a3, a4, w0, w1, w2, w3, w4, bsum)


def _bn_body(cnt, cat_r, st_r, w_r, b_r, out_r):
    mu = st_r[0:1, :] / cnt
    var = st_r[1:2, :] / cnt - mu * mu
    inv = lax.rsqrt(var + 1e-5)
    cat = cat_r[...].astype(jnp.float32)
    out_r[...] = (cat - mu) * inv * w_r[...] + b_r[...]


def _bn_pass(cat, stats, w, b, rt):
    n = cat.shape[0]
    grid = n // rt
    row = lambda i: (i, 0)
    fix = lambda i: (0, 0)
    return pl.pallas_call(
        functools.partial(_bn_body, float(n)),
        grid=(grid,),
        in_specs=[
            pl.BlockSpec((rt, D), row),
            pl.BlockSpec((8, D), fix),
            pl.BlockSpec((1, D), fix),
            pl.BlockSpec((1, D), fix),
        ],
        out_specs=pl.BlockSpec((rt, D), row),
        out_shape=jax.ShapeDtypeStruct((n, D), jnp.float32),
    )(cat, stats, w.reshape(1, D), b.reshape(1, D))


def kernel(x, y, deg_g, deg_lg, pm_pd, edge_index_g, edge_index_lg,
           Wtx, btx, Wtd, btd, Wty, bty, Wt0, bt0, Wt1, bt1,
           Wgy, bgy, Wgd, bgd, Wgx, bgx, Wg0, bg0, Wg1, bg1,
           bnx_w, bnx_b, bny_w, bny_b):
    n = x.shape[0]
    m = y.shape[0]

    src_g, dst_g = edge_index_g[0], edge_index_g[1]
    src_l, dst_l = edge_index_lg[0], edge_index_lg[1]

    zb = 5120
    z1, pmpd_y = _seg_sum([x, y], [src_g, "edge"], dst_g, n, zb,
                          r=2000, g=32, nbuf=3, la=2)
    (z2p,) = _seg_sum([z1], [src_g], dst_g, n, zb,
                      r=4000, g=64, nbuf=4, la=2)
    wb = 12800
    (w1,) = _seg_sum([y], [src_l], dst_l, m, wb,
                     r=2000, g=32, nbuf=3, la=2)
    (w2,) = _seg_sum([w1], [src_l], dst_l, m, wb,
                     r=2000, g=32, nbuf=3, la=2)
    pmpd_x = _gather_rows(x, pm_pd)

    bsx = (btx + btd + bt0 + bt1 + bty).reshape(1, D)
    xcat, xst = _fused_pass(x, deg_g, z1[:n], z2p[:n], pmpd_y[:n],
                            Wtx.T, Wtd.T, Wt0.T, Wt1.T, Wty.T, bsx, 1000)
    xn = _bn_pass(xcat, xst, bnx_w, bnx_b, 1000)

    bsy = (bgy + bgd + bg0 + bg1 + bgx).reshape(1, D)
    ycat, yst = _fused_pass(y, deg_lg, w1, w2, pmpd_x,
                            Wgy.T, Wgd.T, Wg0.T, Wg1.T, Wgx.T, bsy, 2000,
                            out_dtype=jnp.bfloat16)
    yn = _bn_pass(ycat, yst, bny_w, bny_b, 2000)
    return (xn, yn)


# speedup vs baseline: 1.0903x; 1.0704x over previous
import functools

import jax
import jax.numpy as jnp
from jax import lax
from jax.experimental import pallas as pl
from jax.experimental.pallas import tpu as pltpu
from jax.experimental.pallas import tpu_sc as plsc

D = 128
H = 64
NC = 2
NS = 16
G = 32


def _seg_mesh():
    return plsc.VectorSubcoreMesh(core_axis_name="c", subcore_axis_name="s",
                                  num_cores=NC, num_subcores=NS)


R = 2000


def _seg_sum(tables, srcs, dst, num_out, block, r=2000, g=32, nbuf=2, la=1):
    E = dst.shape[0]
    T = len(tables)
    ept = E // NS
    assert E % (NS * 16) == 0 and ept % r == 0
    assert block % (NS * 8) == 0
    assert 1 <= la <= nbuf - 1
    nrounds = ept // r
    nb = -(-num_out // block)
    rpt = block // NS
    out_rows = nb * block
    src_arrs = [s for s in srcs if not isinstance(s, str)]
    NA = len(src_arrs)
    src_map = []
    _na = 0
    for s_ in srcs:
        if isinstance(s_, str):
            src_map.append(None)
        else:
            src_map.append(_na)
            _na += 1
    cap = r + g

    def body(*refs):
        it = iter(refs)
        tabs = [next(it) for _ in range(T)]
        sarr = [next(it) for _ in range(NA)]
        dst_h = next(it)
        zeros_h = next(it)
        outs = [next(it) for _ in range(T)]
        dst_e = next(it)
        src_e = [next(it) for _ in range(NA)]
        stage_rel = next(it)
        stages = [next(it) for _ in range(T)]
        rows_v = [next(it) for _ in range(T)]
        accs = [next(it) for _ in range(T)]
        sem_e = next(it)
        sem_g = [next(it) for _ in range(T)]
        sem_s = [next(it) for _ in range(T)]
        sem_d = [next(it) for _ in range(T)]

        c = lax.axis_index("c")
        s = lax.axis_index("s")
        base_e = s * ept

        def zero_accs():
            for acc in accs:
                pltpu.sync_copy(zeros_h.at[pl.ds(0, rpt)],
                                acc.at[pl.ds(s * rpt, rpt)])

                @pl.when(s == NS - 1)
                def _():
                    pltpu.sync_copy(zeros_h.at[pl.ds(0, 8)],
                                    acc.at[pl.ds(block, 8)])

        zero_accs()

        iota16 = lax.iota(jnp.int32, 16)

        def issue_edges(rr):
            slot = lax.rem(rr, 2)
            pltpu.async_copy(dst_h.at[pl.ds(base_e + rr * r, r)],
                             dst_e.at[pl.ds(slot * r, r)], sem_e)
            for a in range(NA):
                pltpu.async_copy(sarr[a].at[pl.ds(base_e + rr * r, r)],
                                 src_e[a].at[pl.ds(slot * r, r)], sem_e)

        def wait_edges():
            for _ in range(1 + NA):
                pltpu.make_async_copy(dst_h.at[pl.ds(0, r)],
                                      dst_e.at[pl.ds(0, r)], sem_e).wait()

        def issue_gather(t, j, slot):
            idx = stages[t].at[pl.ds(j * g, g)]
            pltpu.async_copy(tabs[t].at[idx], rows_v[t].at[slot], sem_g[t])

        def wait_gather(t):
            pltpu.make_async_copy(tabs[t].at[pl.ds(0, g)],
                                  rows_v[t].at[0], sem_g[t]).wait()

        def issue_scatter(t, j, slot):
            idx = stage_rel.at[pl.ds(j * g, g)]
            pltpu.async_copy(rows_v[t].at[slot], accs[t].at[idx], sem_s[t],
                             add=True)

        def wait_scatter(t):
            pltpu.make_async_copy(rows_v[t].at[0],
                                  accs[t].at[pl.ds(0, g)], sem_s[t]).wait()

        def per_block(b, carry):
            @pl.when(lax.rem(b, NC) == c)
            def _():
                lo = b * block
                issue_edges(0)

                def per_round(rr, carry1):
                    wait_edges()

                    @pl.when(rr + 1 < nrounds)
                    def _():
                        issue_edges(rr + 1)

                    eslot = lax.rem(rr, 2) * r
                    ebase = base_e + rr * r

                    def scan_body(i, wp):
                        d = dst_e[pl.ds(eslot + i * 16, 16)]
                        rel = d - lo
                        m = plsc.bitcast(rel, jnp.uint32) < jnp.uint32(block)
                        plsc.store_compressed(stage_rel.at[pl.ds(wp, 16)],
                                              rel, mask=m)
                        for t in range(T):
                            if src_map[t] is None:
                                sv = (ebase + i * 16) + iota16
                            else:
                                sv = src_e[src_map[t]][
                                    pl.ds(eslot + i * 16, 16)]
                            plsc.store_compressed(
                                stages[t].at[pl.ds(wp, 16)], sv, mask=m)
                        cnt = plsc.all_reduce_population_count(m)
                        return wp + cnt[0]

                    n = lax.fori_loop(0, r // 16, scan_body,
                                      jnp.int32(0))
                    for k in range(g // 16):
                        stage_rel[pl.ds(n + k * 16, 16)] = (
                            jnp.full((16,), block, jnp.int32))
                        for t in range(T):
                            stages[t][pl.ds(n + k * 16, 16)] = iota16

                    @pl.when(rr == 0)
                    def _():
                        @pl.when(b >= NC)
                        def _():
                            for t in range(T):
                                pltpu.make_async_copy(
                                    accs[t].at[pl.ds(0, rpt)],
                                    outs[t].at[pl.ds(0, rpt)],
                                    sem_d[t]).wait()
                            zero_accs()
                        plsc.subcore_barrier()

                    nf = (n + g) // g

                    for k in range(la):
                        @pl.when(nf > k)
                        def _(k=k):
                            for t in range(T):
                                issue_gather(t, k, k % nbuf)

                    def flush_body(j, carry2):
                        slot = lax.rem(j, nbuf)

                        @pl.when(j >= nbuf - la)
                        def _():
                            for t in range(T):
                                wait_scatter(t)

                        @pl.when(j + la < nf)
                        def _():
                            for t in range(T):
                                issue_gather(t, j + la,
                                             lax.rem(j + la, nbuf))
                        for t in range(T):
                            wait_gather(t)
                            issue_scatter(t, j, slot)
                        return carry2

                    lax.fori_loop(0, nf, flush_body, 0)
                    for t in range(T):
                        wait_scatter(t)
                    for k in range(2, nbuf - la + 1):
                        @pl.when(nf >= k)
                        def _(k=k):
                            for t in range(T):
                                wait_scatter(t)
                    return carry1

                lax.fori_loop(0, nrounds, per_round, 0)

                plsc.subcore_barrier()
                for t in range(T):
                    pltpu.async_copy(accs[t].at[pl.ds(s * rpt, rpt)],
                                     outs[t].at[pl.ds(lo + s * rpt, rpt)],
                                     sem_d[t])
            return carry

        lax.fori_loop(0, nb, per_block, 0)
        for t in range(T):
            pltpu.make_async_copy(accs[t].at[pl.ds(0, rpt)],
                                  outs[t].at[pl.ds(0, rpt)], sem_d[t]).wait()

    zeros_h = jnp.zeros((rpt, D), jnp.float32)
    fn = pl.kernel(
        body,
        out_type=[jax.ShapeDtypeStruct((out_rows, D), jnp.float32)
                  for _ in range(T)],
        mesh=_seg_mesh(),
        scratch_types=(
            [pltpu.VMEM((2 * r,), jnp.int32)]
            + [pltpu.VMEM((2 * r,), jnp.int32) for _ in range(NA)]
            + [pltpu.VMEM((cap,), jnp.int32)]
            + [pltpu.VMEM((cap,), jnp.int32) for _ in range(T)]
            + [pltpu.VMEM((nbuf, g, D), jnp.float32) for _ in range(T)]
            + [pltpu.VMEM_SHARED((block + 8, D), jnp.float32)
               for _ in range(T)]
            + [pltpu.SemaphoreType.DMA]
            + [pltpu.SemaphoreType.DMA for _ in range(3 * T)]
        ),
        compiler_params=pltpu.CompilerParams(needs_layout_passes=False),
    )
    out = fn(*tables, *src_arrs, dst, zeros_h)
    return out if isinstance(out, (list, tuple)) else [out]


def _gather_rows(table, idx):
    B = idx.shape[0]
    GG = 200
    per_w = B // (NC * NS)
    assert per_w % GG == 0
    nf = per_w // GG

    def body(table_h, idx_h, out_h, idx_v, rows_v, sem_g, sem_o):
        c = lax.axis_index("c")
        s = lax.axis_index("s")
        wid = s * NC + c
        base = wid * per_w
        pltpu.sync_copy(idx_h.at[pl.ds(base, per_w)], idx_v)

        def issue_gather(j, slot):
            ii = idx_v.at[pl.ds(j * GG, GG)]
            pltpu.async_copy(table_h.at[ii], rows_v.at[slot], sem_g)

        issue_gather(0, 0)

        def step(j, carry):
            slot = lax.rem(j, 2)

            @pl.when(j >= 1)
            def _():
                pltpu.make_async_copy(rows_v.at[0],
                                      out_h.at[pl.ds(0, GG)], sem_o).wait()

            @pl.when(j + 1 < nf)
            def _():
                issue_gather(j + 1, 1 - slot)
            pltpu.make_async_copy(table_h.at[pl.ds(0, GG)],
                                  rows_v.at[0], sem_g).wait()
            pltpu.async_copy(rows_v.at[slot],
                             out_h.at[pl.ds(base + j * GG, GG)], sem_o)
            return carry

        lax.fori_loop(0, nf, step, 0)
        pltpu.make_async_copy(rows_v.at[0],
                              out_h.at[pl.ds(0, GG)], sem_o).wait()

    fn = pl.kernel(
        body,
        out_type=jax.ShapeDtypeStruct((B, D), jnp.float32),
        mesh=_seg_mesh(),
        scratch_types=[
            pltpu.VMEM((per_w,), jnp.int32),
            pltpu.VMEM((2, GG, D), jnp.float32),
            pltpu.SemaphoreType.DMA,
            pltpu.SemaphoreType.DMA,
        ],
        compiler_params=pltpu.CompilerParams(needs_layout_passes=False),
    )
    return fn(table, idx)


def _fused_body(x_r, dg_r, a2_r, a3_r, a4_r, w0, w1, w2, w3, w4, bs,
                out_r, st_r):
    i = pl.program_id(0)
    xb = x_r[...]
    acc = jnp.dot(xb, w0[...], preferred_element_type=jnp.float32)
    acc += jnp.dot(dg_r[...] * xb, w1[...], preferred_element_type=jnp.float32)
    acc += jnp.dot(a2_r[...], w2[...], preferred_element_type=jnp.float32)
    acc += jnp.dot(a3_r[...], w3[...], preferred_element_type=jnp.float32)
    acc += jnp.dot(a4_r[...], w4[...], preferred_element_type=jnp.float32)
    acc += bs[...]
    col = lax.broadcasted_iota(jnp.int32, (1, D), 1)
    cat = jnp.where(col < H, acc, jnp.maximum(acc, 0.0))
    out_r[...] = cat.astype(out_r.dtype)

    @pl.when(i == 0)
    def _():
        st_r[...] = jnp.zeros_like(st_r)

    su = jnp.sum(cat, axis=0, keepdims=True)
    sq = jnp.sum(cat * cat, axis=0, keepdims=True)
    st_r[...] += jnp.concatenate([su, sq, jnp.zeros((6, D), jnp.float32)], 0)


def _fused_pass(x, deg, a2, a3, a4, w0, w1, w2, w3, w4, bsum, rt,
                out_dtype=jnp.float32):
    n = x.shape[0]
    assert n % rt == 0
    grid = n // rt
    row = lambda i: (i, 0)
    fix = lambda i: (0, 0)
    return pl.pallas_call(
        _fused_body,
        grid=(grid,),
        in_specs=[
            pl.BlockSpec((rt, D), row),
            pl.BlockSpec((rt, 1), row),
            pl.BlockSpec((rt, D), row),
            pl.BlockSpec((rt, D), row),
            pl.BlockSpec((rt, D), row),
            pl.BlockSpec((D, D), fix),
            pl.BlockSpec((D, D), fix),
            pl.BlockSpec((D, D), fix),
            pl.BlockSpec((D, D), fix),
            pl.BlockSpec((D, D), fix),
            pl.BlockSpec((1, D), fix),
        ],
        out_specs=[
            pl.BlockSpec((rt, D), row),
            pl.BlockSpec((8, D), fix),
        ],
        out_shape=[
            jax.ShapeDtypeStruct((n, D), out_dtype),
            jax.ShapeDtypeStruct((8, D), jnp.float32),
        ],
    )(x, deg, a2, a3, a4, w0, w1, w2, w3, w4, bsum)


def _bn_body(cnt, cat_r, st_r, w_r, b_r, out_r):
    mu = st_r[0:1, :] / cnt
    var = st_r[1:2, :] / cnt - mu * mu
    inv = lax.rsqrt(var + 1e-5)
    cat = cat_r[...].astype(jnp.float32)
    out_r[...] = (cat - mu) * inv * w_r[...] + b_r[...]


def _bn_pass(cat, stats, w, b, rt):
    n = cat.shape[0]
    grid = n // rt
    row = lambda i: (i, 0)
    fix = lambda i: (0, 0)
    return pl.pallas_call(
        functools.partial(_bn_body, float(n)),
        grid=(grid,),
        in_specs=[
            pl.BlockSpec((rt, D), row),
            pl.BlockSpec((8, D), fix),
            pl.BlockSpec((1, D), fix),
            pl.BlockSpec((1, D), fix),
        ],
        out_specs=pl.BlockSpec((rt, D), row),
        out_shape=jax.ShapeDtypeStruct((n, D), jnp.float32),
    )(cat, stats, w.reshape(1, D), b.reshape(1, D))


def kernel(x, y, deg_g, deg_lg, pm_pd, edge_index_g, edge_index_lg,
           Wtx, btx, Wtd, btd, Wty, bty, Wt0, bt0, Wt1, bt1,
           Wgy, bgy, Wgd, bgd, Wgx, bgx, Wg0, bg0, Wg1, bg1,
           bnx_w, bnx_b, bny_w, bny_b):
    n = x.shape[0]
    m = y.shape[0]

    src_g, dst_g = edge_index_g[0], edge_index_g[1]
    src_l, dst_l = edge_index_lg[0], edge_index_lg[1]

    zb = 5120
    z1, pmpd_y = _seg_sum([x, y], [src_g, "edge"], dst_g, n, zb,
                          r=2000, g=32, nbuf=3, la=2)
    (z2p,) = _seg_sum([z1], [src_g], dst_g, n, zb,
                      r=4000, g=64, nbuf=4, la=2)
    wb = 12800
    (w1,) = _seg_sum([y], [src_l], dst_l, m, wb,
                     r=2000, g=32, nbuf=3, la=2)
    (w2,) = _seg_sum([w1], [src_l], dst_l, m, wb,
                     r=2000, g=32, nbuf=3, la=2)
    pmpd_x = _gather_rows(x, pm_pd)

    bsx = (btx + btd + bt0 + bt1 + bty).reshape(1, D)
    xcat, xst = _fused_pass(x, deg_g, z1[:n], z2p[:n], pmpd_y[:n],
                            Wtx.T, Wtd.T, Wt0.T, Wt1.T, Wty.T, bsx, 1000)
    xn = _bn_pass(xcat, xst, bnx_w, bnx_b, 1000)

    bsy = (bgy + bgd + bg0 + bg1 + bgx).reshape(1, D)
    ycat, yst = _fused_pass(y, deg_lg, w1, w2, pmpd_x,
                            Wgy.T, Wgd.T, Wg0.T, Wg1.T, Wgx.T, bsy, 2000,
                            out_dtype=jnp.bfloat16)
    yn = _bn_pass(ycat, yst, bny_w, bny_b, 2000)
    return (xn, yn)


# speedup vs baseline: 1.1379x; 1.0437x over previous
import functools

import jax
import jax.numpy as jnp
from jax import lax
from jax.experimental import pallas as pl
from jax.experimental.pallas import tpu as pltpu
from jax.experimental.pallas import tpu_sc as plsc

D = 128
H = 64
NC = 2
NS = 16
G = 32


def _seg_mesh():
    return plsc.VectorSubcoreMesh(core_axis_name="c", subcore_axis_name="s",
                                  num_cores=NC, num_subcores=NS)


R = 2000


def _seg_sum(tables, srcs, dst, num_out, block, r=2000, g=32, nbuf=2, la=1):
    E = dst.shape[0]
    T = len(tables)
    ept = E // NS
    assert E % (NS * 16) == 0 and ept % r == 0
    assert block % (NS * 8) == 0
    assert 1 <= la <= nbuf - 1
    nrounds = ept // r
    nb = -(-num_out // block)
    rpt = block // NS
    out_rows = nb * block
    src_arrs = [s for s in srcs if not isinstance(s, str)]
    NA = len(src_arrs)
    src_map = []
    _na = 0
    for s_ in srcs:
        if isinstance(s_, str):
            src_map.append(None)
        else:
            src_map.append(_na)
            _na += 1
    cap = r + g

    def body(*refs):
        it = iter(refs)
        tabs = [next(it) for _ in range(T)]
        sarr = [next(it) for _ in range(NA)]
        dst_h = next(it)
        zeros_h = next(it)
        outs = [next(it) for _ in range(T)]
        dst_e = next(it)
        src_e = [next(it) for _ in range(NA)]
        stage_rel = next(it)
        stages = [next(it) for _ in range(T)]
        rows_v = [next(it) for _ in range(T)]
        accs = [next(it) for _ in range(T)]
        sem_e = next(it)
        sem_g = [next(it) for _ in range(T)]
        sem_s = [next(it) for _ in range(T)]
        sem_d = [next(it) for _ in range(T)]

        c = lax.axis_index("c")
        s = lax.axis_index("s")
        base_e = s * ept

        def zero_accs():
            for acc in accs:
                pltpu.sync_copy(zeros_h.at[pl.ds(0, rpt)],
                                acc.at[pl.ds(s * rpt, rpt)])

                @pl.when(s == NS - 1)
                def _():
                    pltpu.sync_copy(zeros_h.at[pl.ds(0, 8)],
                                    acc.at[pl.ds(block, 8)])

        zero_accs()

        iota16 = lax.iota(jnp.int32, 16)

        def issue_edges(rr):
            slot = lax.rem(rr, 2)
            pltpu.async_copy(dst_h.at[pl.ds(base_e + rr * r, r)],
                             dst_e.at[pl.ds(slot * r, r)], sem_e)
            for a in range(NA):
                pltpu.async_copy(sarr[a].at[pl.ds(base_e + rr * r, r)],
                                 src_e[a].at[pl.ds(slot * r, r)], sem_e)

        def wait_edges():
            for _ in range(1 + NA):
                pltpu.make_async_copy(dst_h.at[pl.ds(0, r)],
                                      dst_e.at[pl.ds(0, r)], sem_e).wait()

        def issue_gather(t, j, slot):
            idx = stages[t].at[pl.ds(j * g, g)]
            pltpu.async_copy(tabs[t].at[idx], rows_v[t].at[slot], sem_g[t])

        def wait_gather(t):
            pltpu.make_async_copy(tabs[t].at[pl.ds(0, g)],
                                  rows_v[t].at[0], sem_g[t]).wait()

        def issue_scatter(t, j, slot):
            idx = stage_rel.at[pl.ds(j * g, g)]
            pltpu.async_copy(rows_v[t].at[slot], accs[t].at[idx], sem_s[t],
                             add=True)

        def wait_scatter(t):
            pltpu.make_async_copy(rows_v[t].at[0],
                                  accs[t].at[pl.ds(0, g)], sem_s[t]).wait()

        def per_block(b, carry):
            @pl.when(lax.rem(b, NC) == c)
            def _():
                lo = b * block
                issue_edges(0)

                def per_round(rr, carry1):
                    wait_edges()

                    @pl.when(rr + 1 < nrounds)
                    def _():
                        issue_edges(rr + 1)

                    eslot = lax.rem(rr, 2) * r
                    ebase = base_e + rr * r

                    def scan_body(i, wp):
                        d = dst_e[pl.ds(eslot + i * 16, 16)]
                        rel = d - lo
                        m = plsc.bitcast(rel, jnp.uint32) < jnp.uint32(block)
                        plsc.store_compressed(stage_rel.at[pl.ds(wp, 16)],
                                              rel, mask=m)
                        for t in range(T):
                            if src_map[t] is None:
                                sv = (ebase + i * 16) + iota16
                            else:
                                sv = src_e[src_map[t]][
                                    pl.ds(eslot + i * 16, 16)]
                            plsc.store_compressed(
                                stages[t].at[pl.ds(wp, 16)], sv, mask=m)
                        cnt = plsc.all_reduce_population_count(m)
                        return wp + cnt[0]

                    n = lax.fori_loop(0, r // 16, scan_body,
                                      jnp.int32(0))
                    for k in range(g // 16):
                        stage_rel[pl.ds(n + k * 16, 16)] = (
                            jnp.full((16,), block, jnp.int32))
                        for t in range(T):
                            stages[t][pl.ds(n + k * 16, 16)] = iota16

                    @pl.when(rr == 0)
                    def _():
                        @pl.when(b >= NC)
                        def _():
                            for t in range(T):
                                pltpu.make_async_copy(
                                    accs[t].at[pl.ds(0, rpt)],
                                    outs[t].at[pl.ds(0, rpt)],
                                    sem_d[t]).wait()
                            zero_accs()
                        plsc.subcore_barrier()

                    nf = (n + g) // g

                    for k in range(la):
                        @pl.when(nf > k)
                        def _(k=k):
                            for t in range(T):
                                issue_gather(t, k, k % nbuf)

                    def flush_body(j, carry2):
                        slot = lax.rem(j, nbuf)

                        @pl.when(j >= nbuf - la)
                        def _():
                            for t in range(T):
                                wait_scatter(t)

                        @pl.when(j + la < nf)
                        def _():
                            for t in range(T):
                                issue_gather(t, j + la,
                                             lax.rem(j + la, nbuf))
                        for t in range(T):
                            wait_gather(t)
                            issue_scatter(t, j, slot)
                        return carry2

                    lax.fori_loop(0, nf, flush_body, 0)
                    for t in range(T):
                        wait_scatter(t)
                    for k in range(2, nbuf - la + 1):
                        @pl.when(nf >= k)
                        def _(k=k):
                            for t in range(T):
                                wait_scatter(t)
                    return carry1

                lax.fori_loop(0, nrounds, per_round, 0)

                plsc.subcore_barrier()
                for t in range(T):
                    pltpu.async_copy(accs[t].at[pl.ds(s * rpt, rpt)],
                                     outs[t].at[pl.ds(lo + s * rpt, rpt)],
                                     sem_d[t])
            return carry

        lax.fori_loop(0, nb, per_block, 0)
        for t in range(T):
            pltpu.make_async_copy(accs[t].at[pl.ds(0, rpt)],
                                  outs[t].at[pl.ds(0, rpt)], sem_d[t]).wait()

    zeros_h = jnp.zeros((rpt, D), jnp.float32)
    fn = pl.kernel(
        body,
        out_type=[jax.ShapeDtypeStruct((out_rows, D), jnp.float32)
                  for _ in range(T)],
        mesh=_seg_mesh(),
        scratch_types=(
            [pltpu.VMEM((2 * r,), jnp.int32)]
            + [pltpu.VMEM((2 * r,), jnp.int32) for _ in range(NA)]
            + [pltpu.VMEM((cap,), jnp.int32)]
            + [pltpu.VMEM((cap,), jnp.int32) for _ in range(T)]
            + [pltpu.VMEM((nbuf, g, D), jnp.float32) for _ in range(T)]
            + [pltpu.VMEM_SHARED((block + 8, D), jnp.float32)
               for _ in range(T)]
            + [pltpu.SemaphoreType.DMA]
            + [pltpu.SemaphoreType.DMA for _ in range(3 * T)]
        ),
        compiler_params=pltpu.CompilerParams(needs_layout_passes=False),
    )
    out = fn(*tables, *src_arrs, dst, zeros_h)
    return out if isinstance(out, (list, tuple)) else [out]


def _gather_rows(table, idx):
    B = idx.shape[0]
    GG = 200
    per_w = B // (NC * NS)
    assert per_w % GG == 0
    nf = per_w // GG

    def body(table_h, idx_h, out_h, idx_v, rows_v, sem_g, sem_o):
        c = lax.axis_index("c")
        s = lax.axis_index("s")
        wid = s * NC + c
        base = wid * per_w
        pltpu.sync_copy(idx_h.at[pl.ds(base, per_w)], idx_v)

        def issue_gather(j, slot):
            ii = idx_v.at[pl.ds(j * GG, GG)]
            pltpu.async_copy(table_h.at[ii], rows_v.at[slot], sem_g)

        issue_gather(0, 0)

        def step(j, carry):
            slot = lax.rem(j, 2)

            @pl.when(j >= 1)
            def _():
                pltpu.make_async_copy(rows_v.at[0],
                                      out_h.at[pl.ds(0, GG)], sem_o).wait()

            @pl.when(j + 1 < nf)
            def _():
                issue_gather(j + 1, 1 - slot)
            pltpu.make_async_copy(table_h.at[pl.ds(0, GG)],
                                  rows_v.at[0], sem_g).wait()
            pltpu.async_copy(rows_v.at[slot],
                             out_h.at[pl.ds(base + j * GG, GG)], sem_o)
            return carry

        lax.fori_loop(0, nf, step, 0)
        pltpu.make_async_copy(rows_v.at[0],
                              out_h.at[pl.ds(0, GG)], sem_o).wait()

    fn = pl.kernel(
        body,
        out_type=jax.ShapeDtypeStruct((B, D), jnp.float32),
        mesh=_seg_mesh(),
        scratch_types=[
            pltpu.VMEM((per_w,), jnp.int32),
            pltpu.VMEM((2, GG, D), jnp.float32),
            pltpu.SemaphoreType.DMA,
            pltpu.SemaphoreType.DMA,
        ],
        compiler_params=pltpu.CompilerParams(needs_layout_passes=False),
    )
    return fn(table, idx)


def _fused_body(x_r, dg_r, a2_r, a3_r, a4_r, w0, w1, w2, w3, w4, bs,
                out_r, st_r):
    i = pl.program_id(0)
    xb = x_r[...]
    acc = jnp.dot(xb, w0[...], preferred_element_type=jnp.float32)
    acc += jnp.dot(dg_r[...] * xb, w1[...], preferred_element_type=jnp.float32)
    acc += jnp.dot(a2_r[...], w2[...], preferred_element_type=jnp.float32)
    acc += jnp.dot(a3_r[...], w3[...], preferred_element_type=jnp.float32)
    acc += jnp.dot(a4_r[...], w4[...], preferred_element_type=jnp.float32)
    acc += bs[...]
    col = lax.broadcasted_iota(jnp.int32, (1, D), 1)
    cat = jnp.where(col < H, acc, jnp.maximum(acc, 0.0))
    out_r[...] = cat.astype(out_r.dtype)

    @pl.when(i == 0)
    def _():
        st_r[...] = jnp.zeros_like(st_r)

    su = jnp.sum(cat, axis=0, keepdims=True)
    sq = jnp.sum(cat * cat, axis=0, keepdims=True)
    st_r[...] += jnp.concatenate([su, sq, jnp.zeros((6, D), jnp.float32)], 0)


def _fused_pass(x, deg, a2, a3, a4, w0, w1, w2, w3, w4, bsum, rt,
                out_dtype=jnp.float32):
    n = x.shape[0]
    assert n % rt == 0
    grid = n // rt
    row = lambda i: (i, 0)
    fix = lambda i: (0, 0)
    return pl.pallas_call(
        _fused_body,
        grid=(grid,),
        in_specs=[
            pl.BlockSpec((rt, D), row),
            pl.BlockSpec((rt, 1), row),
            pl.BlockSpec((rt, D), row),
            pl.BlockSpec((rt, D), row),
            pl.BlockSpec((rt, D), row),
            pl.BlockSpec((D, D), fix),
            pl.BlockSpec((D, D), fix),
            pl.BlockSpec((D, D), fix),
            pl.BlockSpec((D, D), fix),
            pl.BlockSpec((D, D), fix),
            pl.BlockSpec((1, D), fix),
        ],
        out_specs=[
            pl.BlockSpec((rt, D), row),
            pl.BlockSpec((8, D), fix),
        ],
        out_shape=[
            jax.ShapeDtypeStruct((n, D), out_dtype),
            jax.ShapeDtypeStruct((8, D), jnp.float32),
        ],
    )(x, deg, a2, a3, a4, w0, w1, w2, w3, w4, bsum)


def _bn_body(cnt, cat_r, st_r, w_r, b_r, out_r):
    mu = st_r[0:1, :] / cnt
    var = st_r[1:2, :] / cnt - mu * mu
    inv = lax.rsqrt(var + 1e-5)
    cat = cat_r[...].astype(jnp.float32)
    out_r[...] = (cat - mu) * inv * w_r[...] + b_r[...]


def _bn_pass(cat, stats, w, b, rt):
    n = cat.shape[0]
    grid = n // rt
    row = lambda i: (i, 0)
    fix = lambda i: (0, 0)
    return pl.pallas_call(
        functools.partial(_bn_body, float(n)),
        grid=(grid,),
        in_specs=[
            pl.BlockSpec((rt, D), row),
            pl.BlockSpec((8, D), fix),
            pl.BlockSpec((1, D), fix),
            pl.BlockSpec((1, D), fix),
        ],
        out_specs=pl.BlockSpec((rt, D), row),
        out_shape=jax.ShapeDtypeStruct((n, D), jnp.float32),
    )(cat, stats, w.reshape(1, D), b.reshape(1, D))


def kernel(x, y, deg_g, deg_lg, pm_pd, edge_index_g, edge_index_lg,
           Wtx, btx, Wtd, btd, Wty, bty, Wt0, bt0, Wt1, bt1,
           Wgy, bgy, Wgd, bgd, Wgx, bgx, Wg0, bg0, Wg1, bg1,
           bnx_w, bnx_b, bny_w, bny_b):
    n = x.shape[0]
    m = y.shape[0]

    src_g, dst_g = edge_index_g[0], edge_index_g[1]
    src_l, dst_l = edge_index_lg[0], edge_index_lg[1]

    zb = 5120
    z1, pmpd_y = _seg_sum([x, y], [src_g, "edge"], dst_g, n, zb,
                          r=2000, g=32, nbuf=3, la=2)
    (z2p,) = _seg_sum([z1], [src_g], dst_g, n, zb,
                      r=4000, g=64, nbuf=4, la=2)
    wb = 12800
    (w1,) = _seg_sum([y], [src_l], dst_l, m, wb,
                     r=2000, g=32, nbuf=3, la=2)
    (w2,) = _seg_sum([w1], [src_l], dst_l, m, wb,
                     r=2000, g=32, nbuf=3, la=2)
    pmpd_x = _gather_rows(x, pm_pd)

    bsx = (btx + btd + bt0 + bt1 + bty).reshape(1, D)
    xcat, xst = _fused_pass(x, deg_g, z1[:n], z2p[:n], pmpd_y[:n],
                            Wtx.T, Wtd.T, Wt0.T, Wt1.T, Wty.T, bsx, 1000)
    xn = _bn_pass(xcat, xst, bnx_w, bnx_b, 1000)

    bsy = (bgy + bgd + bg0 + bg1 + bgx).reshape(1, D)
    ycat, yst = _fused_pass(y, deg_lg, w1, w2, pmpd_x,
                            Wgy.T, Wgd.T, Wg0.T, Wg1.T, Wgx.T, bsy, 4000,
                            out_dtype=jnp.bfloat16)
    yn = _bn_pass(ycat, yst, bny_w, bny_b, 4000)
    return (xn, yn)


# speedup vs baseline: 1.1519x; 1.0123x over previous
import functools

import jax
import jax.numpy as jnp
from jax import lax
from jax.experimental import pallas as pl
from jax.experimental.pallas import tpu as pltpu
from jax.experimental.pallas import tpu_sc as plsc

D = 128
H = 64
NC = 2
NS = 16
G = 32


def _seg_mesh():
    return plsc.VectorSubcoreMesh(core_axis_name="c", subcore_axis_name="s",
                                  num_cores=NC, num_subcores=NS)


R = 2000


def _seg_sum(tables, srcs, dst, num_out, block, r=2000, g=32, nbuf=2, la=1):
    E = dst.shape[0]
    T = len(tables)
    ept = E // NS
    assert E % (NS * 16) == 0 and ept % r == 0
    assert block % (NS * 8) == 0
    assert 1 <= la <= nbuf - 1
    nrounds = ept // r
    nb = -(-num_out // block)
    rpt = block // NS
    out_rows = nb * block
    src_arrs = [s for s in srcs if not isinstance(s, str)]
    NA = len(src_arrs)
    src_map = []
    _na = 0
    for s_ in srcs:
        if isinstance(s_, str):
            src_map.append(None)
        else:
            src_map.append(_na)
            _na += 1
    cap = r + g

    def body(*refs):
        it = iter(refs)
        tabs = [next(it) for _ in range(T)]
        sarr = [next(it) for _ in range(NA)]
        dst_h = next(it)
        zeros_h = next(it)
        outs = [next(it) for _ in range(T)]
        dst_e = next(it)
        src_e = [next(it) for _ in range(NA)]
        stage_rel = next(it)
        stages = [next(it) for _ in range(T)]
        rows_v = [next(it) for _ in range(T)]
        accs = [next(it) for _ in range(T)]
        sem_e = next(it)
        sem_g = [next(it) for _ in range(T)]
        sem_s = [next(it) for _ in range(T)]
        sem_d = [next(it) for _ in range(T)]

        c = lax.axis_index("c")
        s = lax.axis_index("s")
        base_e = s * ept

        def zero_accs():
            for acc in accs:
                pltpu.sync_copy(zeros_h.at[pl.ds(0, rpt)],
                                acc.at[pl.ds(s * rpt, rpt)])

                @pl.when(s == NS - 1)
                def _():
                    pltpu.sync_copy(zeros_h.at[pl.ds(0, 8)],
                                    acc.at[pl.ds(block, 8)])

        zero_accs()

        iota16 = lax.iota(jnp.int32, 16)

        def issue_edges(rr):
            slot = lax.rem(rr, 2)
            pltpu.async_copy(dst_h.at[pl.ds(base_e + rr * r, r)],
                             dst_e.at[pl.ds(slot * r, r)], sem_e)
            for a in range(NA):
                pltpu.async_copy(sarr[a].at[pl.ds(base_e + rr * r, r)],
                                 src_e[a].at[pl.ds(slot * r, r)], sem_e)

        def wait_edges():
            for _ in range(1 + NA):
                pltpu.make_async_copy(dst_h.at[pl.ds(0, r)],
                                      dst_e.at[pl.ds(0, r)], sem_e).wait()

        def issue_gather(t, j, slot):
            idx = stages[t].at[pl.ds(j * g, g)]
            pltpu.async_copy(tabs[t].at[idx], rows_v[t].at[slot], sem_g[t])

        def wait_gather(t):
            pltpu.make_async_copy(tabs[t].at[pl.ds(0, g)],
                                  rows_v[t].at[0], sem_g[t]).wait()

        def issue_scatter(t, j, slot):
            idx = stage_rel.at[pl.ds(j * g, g)]
            pltpu.async_copy(rows_v[t].at[slot], accs[t].at[idx], sem_s[t],
                             add=True)

        def wait_scatter(t):
            pltpu.make_async_copy(rows_v[t].at[0],
                                  accs[t].at[pl.ds(0, g)], sem_s[t]).wait()

        def per_block(b, carry):
            @pl.when(lax.rem(b, NC) == c)
            def _():
                lo = b * block
                issue_edges(0)

                def per_round(rr, carry1):
                    wait_edges()

                    @pl.when(rr + 1 < nrounds)
                    def _():
                        issue_edges(rr + 1)

                    eslot = lax.rem(rr, 2) * r
                    ebase = base_e + rr * r

                    def scan_body(i, wp):
                        d = dst_e[pl.ds(eslot + i * 16, 16)]
                        rel = d - lo
                        m = plsc.bitcast(rel, jnp.uint32) < jnp.uint32(block)
                        plsc.store_compressed(stage_rel.at[pl.ds(wp, 16)],
                                              rel, mask=m)
                        for t in range(T):
                            if src_map[t] is None:
                                sv = (ebase + i * 16) + iota16
                            else:
                                sv = src_e[src_map[t]][
                                    pl.ds(eslot + i * 16, 16)]
                            plsc.store_compressed(
                                stages[t].at[pl.ds(wp, 16)], sv, mask=m)
                        cnt = plsc.all_reduce_population_count(m)
                        return wp + cnt[0]

                    n = lax.fori_loop(0, r // 16, scan_body,
                                      jnp.int32(0))
                    for k in range(g // 16):
                        stage_rel[pl.ds(n + k * 16, 16)] = (
                            jnp.full((16,), block, jnp.int32))
                        for t in range(T):
                            stages[t][pl.ds(n + k * 16, 16)] = iota16

                    @pl.when(rr == 0)
                    def _():
                        @pl.when(b >= NC)
                        def _():
                            for t in range(T):
                                pltpu.make_async_copy(
                                    accs[t].at[pl.ds(0, rpt)],
                                    outs[t].at[pl.ds(0, rpt)],
                                    sem_d[t]).wait()
                            zero_accs()
                        plsc.subcore_barrier()

                    nf = (n + g) // g

                    for k in range(la):
                        @pl.when(nf > k)
                        def _(k=k):
                            for t in range(T):
                                issue_gather(t, k, k % nbuf)

                    def flush_body(j, carry2):
                        slot = lax.rem(j, nbuf)

                        @pl.when(j >= nbuf - la)
                        def _():
                            for t in range(T):
                                wait_scatter(t)

                        @pl.when(j + la < nf)
                        def _():
                            for t in range(T):
                                issue_gather(t, j + la,
                                             lax.rem(j + la, nbuf))
                        for t in range(T):
                            wait_gather(t)
                            issue_scatter(t, j, slot)
                        return carry2

                    lax.fori_loop(0, nf, flush_body, 0)
                    for t in range(T):
                        wait_scatter(t)
                    for k in range(2, nbuf - la + 1):
                        @pl.when(nf >= k)
                        def _(k=k):
                            for t in range(T):
                                wait_scatter(t)
                    return carry1

                lax.fori_loop(0, nrounds, per_round, 0)

                plsc.subcore_barrier()
                for t in range(T):
                    pltpu.async_copy(accs[t].at[pl.ds(s * rpt, rpt)],
                                     outs[t].at[pl.ds(lo + s * rpt, rpt)],
                                     sem_d[t])
            return carry

        lax.fori_loop(0, nb, per_block, 0)
        for t in range(T):
            pltpu.make_async_copy(accs[t].at[pl.ds(0, rpt)],
                                  outs[t].at[pl.ds(0, rpt)], sem_d[t]).wait()

    zeros_h = jnp.zeros((rpt, D), jnp.float32)
    fn = pl.kernel(
        body,
        out_type=[jax.ShapeDtypeStruct((out_rows, D), jnp.float32)
                  for _ in range(T)],
        mesh=_seg_mesh(),
        scratch_types=(
            [pltpu.VMEM((2 * r,), jnp.int32)]
            + [pltpu.VMEM((2 * r,), jnp.int32) for _ in range(NA)]
            + [pltpu.VMEM((cap,), jnp.int32)]
            + [pltpu.VMEM((cap,), jnp.int32) for _ in range(T)]
            + [pltpu.VMEM((nbuf, g, D), jnp.float32) for _ in range(T)]
            + [pltpu.VMEM_SHARED((block + 8, D), jnp.float32)
               for _ in range(T)]
            + [pltpu.SemaphoreType.DMA]
            + [pltpu.SemaphoreType.DMA for _ in range(3 * T)]
        ),
        compiler_params=pltpu.CompilerParams(needs_layout_passes=False),
    )
    out = fn(*tables, *src_arrs, dst, zeros_h)
    return out if isinstance(out, (list, tuple)) else [out]


def _gather_rows(table, idx):
    B = idx.shape[0]
    GG = 200
    per_w = B // (NC * NS)
    assert per_w % GG == 0
    nf = per_w // GG

    def body(table_h, idx_h, out_h, idx_v, rows_v, sem_g, sem_o):
        c = lax.axis_index("c")
        s = lax.axis_index("s")
        wid = s * NC + c
        base = wid * per_w
        pltpu.sync_copy(idx_h.at[pl.ds(base, per_w)], idx_v)

        def issue_gather(j, slot):
            ii = idx_v.at[pl.ds(j * GG, GG)]
            pltpu.async_copy(table_h.at[ii], rows_v.at[slot], sem_g)

        issue_gather(0, 0)

        def step(j, carry):
            slot = lax.rem(j, 2)

            @pl.when(j >= 1)
            def _():
                pltpu.make_async_copy(rows_v.at[0],
                                      out_h.at[pl.ds(0, GG)], sem_o).wait()

            @pl.when(j + 1 < nf)
            def _():
                issue_gather(j + 1, 1 - slot)
            pltpu.make_async_copy(table_h.at[pl.ds(0, GG)],
                                  rows_v.at[0], sem_g).wait()
            pltpu.async_copy(rows_v.at[slot],
                             out_h.at[pl.ds(base + j * GG, GG)], sem_o)
            return carry

        lax.fori_loop(0, nf, step, 0)
        pltpu.make_async_copy(rows_v.at[0],
                              out_h.at[pl.ds(0, GG)], sem_o).wait()

    fn = pl.kernel(
        body,
        out_type=jax.ShapeDtypeStruct((B, D), jnp.float32),
        mesh=_seg_mesh(),
        scratch_types=[
            pltpu.VMEM((per_w,), jnp.int32),
            pltpu.VMEM((2, GG, D), jnp.float32),
            pltpu.SemaphoreType.DMA,
            pltpu.SemaphoreType.DMA,
        ],
        compiler_params=pltpu.CompilerParams(needs_layout_passes=False),
    )
    return fn(table, idx)


def _fused_body(x_r, dg_r, a2_r, a3_r, a4_r, w0, w1, w2, w3, w4, bs,
                out_r, st_r):
    i = pl.program_id(0)
    xb = x_r[...]
    acc = jnp.dot(xb, w0[...], preferred_element_type=jnp.float32)
    acc += jnp.dot(dg_r[...] * xb, w1[...], preferred_element_type=jnp.float32)
    acc += jnp.dot(a2_r[...], w2[...], preferred_element_type=jnp.float32)
    acc += jnp.dot(a3_r[...], w3[...], preferred_element_type=jnp.float32)
    acc += jnp.dot(a4_r[...], w4[...], preferred_element_type=jnp.float32)
    acc += bs[...]
    col = lax.broadcasted_iota(jnp.int32, (1, D), 1)
    cat = jnp.where(col < H, acc, jnp.maximum(acc, 0.0))
    out_r[...] = cat.astype(out_r.dtype)

    @pl.when(i == 0)
    def _():
        st_r[...] = jnp.zeros_like(st_r)

    su = jnp.sum(cat, axis=0, keepdims=True)
    sq = jnp.sum(cat * cat, axis=0, keepdims=True)
    st_r[...] += jnp.concatenate([su, sq, jnp.zeros((6, D), jnp.float32)], 0)


def _fused_pass(x, deg, a2, a3, a4, w0, w1, w2, w3, w4, bsum, rt,
                out_dtype=jnp.float32):
    n = x.shape[0]
    assert n % rt == 0
    grid = n // rt
    row = lambda i: (i, 0)
    fix = lambda i: (0, 0)
    return pl.pallas_call(
        _fused_body,
        grid=(grid,),
        in_specs=[
            pl.BlockSpec((rt, D), row),
            pl.BlockSpec((rt, 1), row),
            pl.BlockSpec((rt, D), row),
            pl.BlockSpec((rt, D), row),
            pl.BlockSpec((rt, D), row),
            pl.BlockSpec((D, D), fix),
            pl.BlockSpec((D, D), fix),
            pl.BlockSpec((D, D), fix),
            pl.BlockSpec((D, D), fix),
            pl.BlockSpec((D, D), fix),
            pl.BlockSpec((1, D), fix),
        ],
        out_specs=[
            pl.BlockSpec((rt, D), row),
            pl.BlockSpec((8, D), fix),
        ],
        out_shape=[
            jax.ShapeDtypeStruct((n, D), out_dtype),
            jax.ShapeDtypeStruct((8, D), jnp.float32),
        ],
    )(x, deg, a2, a3, a4, w0, w1, w2, w3, w4, bsum)


def _bn_body(cnt, cat_r, st_r, w_r, b_r, out_r):
    mu = st_r[0:1, :] / cnt
    var = st_r[1:2, :] / cnt - mu * mu
    inv = lax.rsqrt(var + 1e-5)
    cat = cat_r[...].astype(jnp.float32)
    out_r[...] = (cat - mu) * inv * w_r[...] + b_r[...]


def _bn_pass(cat, stats, w, b, rt):
    n = cat.shape[0]
    grid = n // rt
    row = lambda i: (i, 0)
    fix = lambda i: (0, 0)
    return pl.pallas_call(
        functools.partial(_bn_body, float(n)),
        grid=(grid,),
        in_specs=[
            pl.BlockSpec((rt, D), row),
            pl.BlockSpec((8, D), fix),
            pl.BlockSpec((1, D), fix),
            pl.BlockSpec((1, D), fix),
        ],
        out_specs=pl.BlockSpec((rt, D), row),
        out_shape=jax.ShapeDtypeStruct((n, D), jnp.float32),
    )(cat, stats, w.reshape(1, D), b.reshape(1, D))


def kernel(x, y, deg_g, deg_lg, pm_pd, edge_index_g, edge_index_lg,
           Wtx, btx, Wtd, btd, Wty, bty, Wt0, bt0, Wt1, bt1,
           Wgy, bgy, Wgd, bgd, Wgx, bgx, Wg0, bg0, Wg1, bg1,
           bnx_w, bnx_b, bny_w, bny_b):
    n = x.shape[0]
    m = y.shape[0]

    src_g, dst_g = edge_index_g[0], edge_index_g[1]
    src_l, dst_l = edge_index_lg[0], edge_index_lg[1]

    zb = 5120
    z1, pmpd_y = _seg_sum([x, y], [src_g, "edge"], dst_g, n, zb,
                          r=2000, g=32, nbuf=3, la=2)
    (z2p,) = _seg_sum([z1], [src_g], dst_g, n, zb,
                      r=4000, g=64, nbuf=4, la=2)
    wb = 12800
    (w1,) = _seg_sum([y], [src_l], dst_l, m, wb,
                     r=2000, g=32, nbuf=3, la=2)
    (w2,) = _seg_sum([w1], [src_l], dst_l, m, wb,
                     r=2000, g=32, nbuf=3, la=2)
    pmpd_x = _gather_rows(x, pm_pd)

    bsx = (btx + btd + bt0 + bt1 + bty).reshape(1, D)
    xcat, xst = _fused_pass(x, deg_g, z1[:n], z2p[:n], pmpd_y[:n],
                            Wtx.T, Wtd.T, Wt0.T, Wt1.T, Wty.T, bsx, 2000)
    xn = _bn_pass(xcat, xst, bnx_w, bnx_b, 2000)

    bsy = (bgy + bgd + bg0 + bg1 + bgx).reshape(1, D)
    ycat, yst = _fused_pass(y, deg_lg, w1, w2, pmpd_x,
                            Wgy.T, Wgd.T, Wg0.T, Wg1.T, Wgx.T, bsy, 8000,
                            out_dtype=jnp.bfloat16)
    yn = _bn_pass(ycat, yst, bny_w, bny_b, 8000)
    return (xn, yn)


# speedup vs baseline: 1.1522x; 1.0003x over previous
import functools

import jax
import jax.numpy as jnp
from jax import lax
from jax.experimental import pallas as pl
from jax.experimental.pallas import tpu as pltpu
from jax.experimental.pallas import tpu_sc as plsc

D = 128
H = 64
NC = 2
NS = 16
G = 32


def _seg_mesh():
    return plsc.VectorSubcoreMesh(core_axis_name="c", subcore_axis_name="s",
                                  num_cores=NC, num_subcores=NS)


R = 2000


def _seg_sum(tables, srcs, dst, num_out, block, r=2000, g=32, nbuf=2, la=1):
    E = dst.shape[0]
    T = len(tables)
    ept = E // NS
    assert E % (NS * 16) == 0 and ept % r == 0
    assert block % (NS * 8) == 0
    assert 1 <= la <= nbuf - 1
    nrounds = ept // r
    nb = -(-num_out // block)
    rpt = block // NS
    out_rows = nb * block
    src_arrs = [s for s in srcs if not isinstance(s, str)]
    NA = len(src_arrs)
    src_map = []
    _na = 0
    for s_ in srcs:
        if isinstance(s_, str):
            src_map.append(None)
        else:
            src_map.append(_na)
            _na += 1
    cap = r + g

    def body(*refs):
        it = iter(refs)
        tabs = [next(it) for _ in range(T)]
        sarr = [next(it) for _ in range(NA)]
        dst_h = next(it)
        zeros_h = next(it)
        outs = [next(it) for _ in range(T)]
        dst_e = next(it)
        src_e = [next(it) for _ in range(NA)]
        stage_rel = next(it)
        stages = [next(it) for _ in range(T)]
        rows_v = [next(it) for _ in range(T)]
        accs = [next(it) for _ in range(T)]
        sem_e = next(it)
        sem_g = [next(it) for _ in range(T)]
        sem_s = [next(it) for _ in range(T)]
        sem_d = [next(it) for _ in range(T)]

        c = lax.axis_index("c")
        s = lax.axis_index("s")
        base_e = s * ept

        def zero_accs():
            for acc in accs:
                pltpu.sync_copy(zeros_h.at[pl.ds(0, rpt)],
                                acc.at[pl.ds(s * rpt, rpt)])

                @pl.when(s == NS - 1)
                def _():
                    pltpu.sync_copy(zeros_h.at[pl.ds(0, 8)],
                                    acc.at[pl.ds(block, 8)])

        zero_accs()

        iota16 = lax.iota(jnp.int32, 16)

        def issue_edges(rr):
            slot = lax.rem(rr, 2)
            pltpu.async_copy(dst_h.at[pl.ds(base_e + rr * r, r)],
                             dst_e.at[pl.ds(slot * r, r)], sem_e)
            for a in range(NA):
                pltpu.async_copy(sarr[a].at[pl.ds(base_e + rr * r, r)],
                                 src_e[a].at[pl.ds(slot * r, r)], sem_e)

        def wait_edges():
            for _ in range(1 + NA):
                pltpu.make_async_copy(dst_h.at[pl.ds(0, r)],
                                      dst_e.at[pl.ds(0, r)], sem_e).wait()

        def issue_gather(t, j, slot):
            idx = stages[t].at[pl.ds(j * g, g)]
            pltpu.async_copy(tabs[t].at[idx], rows_v[t].at[slot], sem_g[t])

        def wait_gather(t):
            pltpu.make_async_copy(tabs[t].at[pl.ds(0, g)],
                                  rows_v[t].at[0], sem_g[t]).wait()

        def issue_scatter(t, j, slot):
            idx = stage_rel.at[pl.ds(j * g, g)]
            pltpu.async_copy(rows_v[t].at[slot], accs[t].at[idx], sem_s[t],
                             add=True)

        def wait_scatter(t):
            pltpu.make_async_copy(rows_v[t].at[0],
                                  accs[t].at[pl.ds(0, g)], sem_s[t]).wait()

        def per_block(b, carry):
            @pl.when(lax.rem(b, NC) == c)
            def _():
                lo = b * block
                issue_edges(0)

                def per_round(rr, carry1):
                    wait_edges()

                    @pl.when(rr + 1 < nrounds)
                    def _():
                        issue_edges(rr + 1)

                    eslot = lax.rem(rr, 2) * r
                    ebase = base_e + rr * r

                    def scan_body(i, wp):
                        d = dst_e[pl.ds(eslot + i * 16, 16)]
                        rel = d - lo
                        m = plsc.bitcast(rel, jnp.uint32) < jnp.uint32(block)
                        plsc.store_compressed(stage_rel.at[pl.ds(wp, 16)],
                                              rel, mask=m)
                        for t in range(T):
                            if src_map[t] is None:
                                sv = (ebase + i * 16) + iota16
                            else:
                                sv = src_e[src_map[t]][
                                    pl.ds(eslot + i * 16, 16)]
                            plsc.store_compressed(
                                stages[t].at[pl.ds(wp, 16)], sv, mask=m)
                        cnt = plsc.all_reduce_population_count(m)
                        return wp + cnt[0]

                    n = lax.fori_loop(0, r // 16, scan_body,
                                      jnp.int32(0))
                    for k in range(g // 16):
                        stage_rel[pl.ds(n + k * 16, 16)] = (
                            jnp.full((16,), block, jnp.int32))
                        for t in range(T):
                            stages[t][pl.ds(n + k * 16, 16)] = iota16

                    @pl.when(rr == 0)
                    def _():
                        @pl.when(b >= NC)
                        def _():
                            for t in range(T):
                                pltpu.make_async_copy(
                                    accs[t].at[pl.ds(0, rpt)],
                                    outs[t].at[pl.ds(0, rpt)],
                                    sem_d[t]).wait()
                            zero_accs()
                        plsc.subcore_barrier()

                    nf = (n + g) // g

                    for k in range(la):
                        @pl.when(nf > k)
                        def _(k=k):
                            for t in range(T):
                                issue_gather(t, k, k % nbuf)

                    def flush_body(j, carry2):
                        slot = lax.rem(j, nbuf)

                        @pl.when(j >= nbuf - la)
                        def _():
                            for t in range(T):
                                wait_scatter(t)

                        @pl.when(j + la < nf)
                        def _():
                            for t in range(T):
                                issue_gather(t, j + la,
                                             lax.rem(j + la, nbuf))
                        for t in range(T):
                            wait_gather(t)
                            issue_scatter(t, j, slot)
                        return carry2

                    lax.fori_loop(0, nf, flush_body, 0)
                    for t in range(T):
                        wait_scatter(t)
                    for k in range(2, nbuf - la + 1):
                        @pl.when(nf >= k)
                        def _(k=k):
                            for t in range(T):
                                wait_scatter(t)
                    return carry1

                lax.fori_loop(0, nrounds, per_round, 0)

                plsc.subcore_barrier()
                for t in range(T):
                    pltpu.async_copy(accs[t].at[pl.ds(s * rpt, rpt)],
                                     outs[t].at[pl.ds(lo + s * rpt, rpt)],
                                     sem_d[t])
            return carry

        lax.fori_loop(0, nb, per_block, 0)
        for t in range(T):
            pltpu.make_async_copy(accs[t].at[pl.ds(0, rpt)],
                                  outs[t].at[pl.ds(0, rpt)], sem_d[t]).wait()

    zeros_h = jnp.zeros((rpt, D), jnp.float32)
    fn = pl.kernel(
        body,
        out_type=[jax.ShapeDtypeStruct((out_rows, D), jnp.float32)
                  for _ in range(T)],
        mesh=_seg_mesh(),
        scratch_types=(
            [pltpu.VMEM((2 * r,), jnp.int32)]
            + [pltpu.VMEM((2 * r,), jnp.int32) for _ in range(NA)]
            + [pltpu.VMEM((cap,), jnp.int32)]
            + [pltpu.VMEM((cap,), jnp.int32) for _ in range(T)]
            + [pltpu.VMEM((nbuf, g, D), jnp.float32) for _ in range(T)]
            + [pltpu.VMEM_SHARED((block + 8, D), jnp.float32)
               for _ in range(T)]
            + [pltpu.SemaphoreType.DMA]
            + [pltpu.SemaphoreType.DMA for _ in range(3 * T)]
        ),
        compiler_params=pltpu.CompilerParams(needs_layout_passes=False),
    )
    out = fn(*tables, *src_arrs, dst, zeros_h)
    return out if isinstance(out, (list, tuple)) else [out]


def _gather_rows(table, idx):
    B = idx.shape[0]
    GG = 200
    per_w = B // (NC * NS)
    assert per_w % GG == 0
    nf = per_w // GG

    def body(table_h, idx_h, out_h, idx_v, rows_v, sem_g, sem_o):
        c = lax.axis_index("c")
        s = lax.axis_index("s")
        wid = s * NC + c
        base = wid * per_w
        pltpu.sync_copy(idx_h.at[pl.ds(base, per_w)], idx_v)

        def issue_gather(j, slot):
            ii = idx_v.at[pl.ds(j * GG, GG)]
            pltpu.async_copy(table_h.at[ii], rows_v.at[slot], sem_g)

        issue_gather(0, 0)
        issue_gather(1, 1)

        def step(j, carry):
            slot = lax.rem(j, 3)

            @pl.when(j >= 1)
            def _():
                pltpu.make_async_copy(rows_v.at[0],
                                      out_h.at[pl.ds(0, GG)], sem_o).wait()

            @pl.when(j + 2 < nf)
            def _():
                issue_gather(j + 2, lax.rem(j + 2, 3))
            pltpu.make_async_copy(table_h.at[pl.ds(0, GG)],
                                  rows_v.at[0], sem_g).wait()
            pltpu.async_copy(rows_v.at[slot],
                             out_h.at[pl.ds(base + j * GG, GG)], sem_o)
            return carry

        lax.fori_loop(0, nf, step, 0)
        pltpu.make_async_copy(rows_v.at[0],
                              out_h.at[pl.ds(0, GG)], sem_o).wait()

    fn = pl.kernel(
        body,
        out_type=jax.ShapeDtypeStruct((B, D), jnp.float32),
        mesh=_seg_mesh(),
        scratch_types=[
            pltpu.VMEM((per_w,), jnp.int32),
            pltpu.VMEM((3, GG, D), jnp.float32),
            pltpu.SemaphoreType.DMA,
            pltpu.SemaphoreType.DMA,
        ],
        compiler_params=pltpu.CompilerParams(needs_layout_passes=False),
    )
    return fn(table, idx)


def _fused_body(x_r, dg_r, a2_r, a3_r, a4_r, w0, w1, w2, w3, w4, bs,
                out_r, st_r):
    i = pl.program_id(0)
    xb = x_r[...]
    acc = jnp.dot(xb, w0[...], preferred_element_type=jnp.float32)
    acc += jnp.dot(dg_r[...] * xb, w1[...], preferred_element_type=jnp.float32)
    acc += jnp.dot(a2_r[...], w2[...], preferred_element_type=jnp.float32)
    acc += jnp.dot(a3_r[...], w3[...], preferred_element_type=jnp.float32)
    acc += jnp.dot(a4_r[...], w4[...], preferred_element_type=jnp.float32)
    acc += bs[...]
    col = lax.broadcasted_iota(jnp.int32, (1, D), 1)
    cat = jnp.where(col < H, acc, jnp.maximum(acc, 0.0))
    out_r[...] = cat.astype(out_r.dtype)

    @pl.when(i == 0)
    def _():
        st_r[...] = jnp.zeros_like(st_r)

    su = jnp.sum(cat, axis=0, keepdims=True)
    sq = jnp.sum(cat * cat, axis=0, keepdims=True)
    st_r[...] += jnp.concatenate([su, sq, jnp.zeros((6, D), jnp.float32)], 0)


def _fused_pass(x, deg, a2, a3, a4, w0, w1, w2, w3, w4, bsum, rt,
                out_dtype=jnp.float32):
    n = x.shape[0]
    assert n % rt == 0
    grid = n // rt
    row = lambda i: (i, 0)
    fix = lambda i: (0, 0)
    return pl.pallas_call(
        _fused_body,
        grid=(grid,),
        in_specs=[
            pl.BlockSpec((rt, D), row),
            pl.BlockSpec((rt, 1), row),
            pl.BlockSpec((rt, D), row),
            pl.BlockSpec((rt, D), row),
            pl.BlockSpec((rt, D), row),
            pl.BlockSpec((D, D), fix),
            pl.BlockSpec((D, D), fix),
            pl.BlockSpec((D, D), fix),
            pl.BlockSpec((D, D), fix),
            pl.BlockSpec((D, D), fix),
            pl.BlockSpec((1, D), fix),
        ],
        out_specs=[
            pl.BlockSpec((rt, D), row),
            pl.BlockSpec((8, D), fix),
        ],
        out_shape=[
            jax.ShapeDtypeStruct((n, D), out_dtype),
            jax.ShapeDtypeStruct((8, D), jnp.float32),
        ],
    )(x, deg, a2, a3, a4, w0, w1, w2, w3, w4, bsum)


def _bn_body(cnt, cat_r, st_r, w_r, b_r, out_r):
    mu = st_r[0:1, :] / cnt
    var = st_r[1:2, :] / cnt - mu * mu
    inv = lax.rsqrt(var + 1e-5)
    cat = cat_r[...].astype(jnp.float32)
    out_r[...] = (cat - mu) * inv * w_r[...] + b_r[...]


def _bn_pass(cat, stats, w, b, rt):
    n = cat.shape[0]
    grid = n // rt
    row = lambda i: (i, 0)
    fix = lambda i: (0, 0)
    return pl.pallas_call(
        functools.partial(_bn_body, float(n)),
        grid=(grid,),
        in_specs=[
            pl.BlockSpec((rt, D), row),
            pl.BlockSpec((8, D), fix),
            pl.BlockSpec((1, D), fix),
            pl.BlockSpec((1, D), fix),
        ],
        out_specs=pl.BlockSpec((rt, D), row),
        out_shape=jax.ShapeDtypeStruct((n, D), jnp.float32),
    )(cat, stats, w.reshape(1, D), b.reshape(1, D))


def kernel(x, y, deg_g, deg_lg, pm_pd, edge_index_g, edge_index_lg,
           Wtx, btx, Wtd, btd, Wty, bty, Wt0, bt0, Wt1, bt1,
           Wgy, bgy, Wgd, bgd, Wgx, bgx, Wg0, bg0, Wg1, bg1,
           bnx_w, bnx_b, bny_w, bny_b):
    n = x.shape[0]
    m = y.shape[0]

    src_g, dst_g = edge_index_g[0], edge_index_g[1]
    src_l, dst_l = edge_index_lg[0], edge_index_lg[1]

    zb = 5120
    z1, pmpd_y = _seg_sum([x, y], [src_g, "edge"], dst_g, n, zb,
                          r=2000, g=32, nbuf=3, la=2)
    (z2p,) = _seg_sum([z1], [src_g], dst_g, n, zb,
                      r=4000, g=64, nbuf=4, la=2)
    wb = 12800
    (w1,) = _seg_sum([y], [src_l], dst_l, m, wb,
                     r=2000, g=32, nbuf=3, la=2)
    (w2,) = _seg_sum([w1], [src_l], dst_l, m, wb,
                     r=2000, g=32, nbuf=3, la=2)
    pmpd_x = _gather_rows(x, pm_pd)

    bsx = (btx + btd + bt0 + bt1 + bty).reshape(1, D)
    xcat, xst = _fused_pass(x, deg_g, z1[:n], z2p[:n], pmpd_y[:n],
                            Wtx.T, Wtd.T, Wt0.T, Wt1.T, Wty.T, bsx, 2000)
    xn = _bn_pass(xcat, xst, bnx_w, bnx_b, 2000)

    bsy = (bgy + bgd + bg0 + bg1 + bgx).reshape(1, D)
    ycat, yst = _fused_pass(y, deg_lg, w1, w2, pmpd_x,
                            Wgy.T, Wgd.T, Wg0.T, Wg1.T, Wgx.T, bsy, 8000,
                            out_dtype=jnp.bfloat16)
    yn = _bn_pass(ycat, yst, bny_w, bny_b, 8000)
    return (xn, yn)
